# Initial kernel scaffold; baseline (speedup 1.0000x reference)
import jax
import jax.numpy as jnp
from jax.experimental import pallas as pl


def kernel(x_drug, edge_drug, inc_drug, x_prot, prot_inc, dp_edge_idx, Wn_d, We_d, Wn_p, Wbd, Wbp, Wq_d, Wk_p, Wv_p, Wq_p, Wk_d, Wv_d, W1, b1, W2):
    raise NotImplementedError("write your pallas kernel here")


# speedup vs baseline: 1.5437x; 1.5437x over previous
import functools

import jax
import jax.numpy as jnp
from jax import lax
from jax.experimental import pallas as pl
from jax.experimental.pallas import tpu as pltpu
from jax.experimental.pallas import tpu_sc as plsc

N_D = 10000
M_D = 2500
N_P = 10000
M_P = 2500
E = 320000
DIN = 128
HID = 64
ROUNDS = 3
SCALE = 1.0 / (HID ** 0.5)

NC = 2
NS = 16
NW = NC * NS
CHUNK = 128
EPW = E // NW
NCHUNK = (EPW + CHUNK - 1) // CHUNK
PAD_EPW = NCHUNK * CHUNK


def _acc_rows(n):
    return ((n + 1 + 255) // 256) * 256


ACC_N = _acc_rows(N_D)
ACC_M = _acc_rows(M_D)


def _mesh():
    return plsc.VectorSubcoreMesh(
        core_axis_name="c", subcore_axis_name="s", num_cores=NC, num_subcores=NS
    )


def _zero_fill(zbuf, nrow, ncol):
    zf = jnp.zeros((16,), jnp.float32)
    cpr = ncol // 16

    def zb(i, _):
        r = i // cpr
        c = (i % cpr) * 16
        zbuf[r, pl.ds(c, 16)] = zf
        return 0

    lax.fori_loop(0, nrow * cpr, zb, 0)


def _zero_fill_1d(zvec, n):
    zf = jnp.zeros((16,), jnp.float32)

    def zb(i, _):
        zvec[pl.ds(i * 16, 16)] = zf
        return 0

    lax.fori_loop(0, n // 16, zb, 0)


@functools.lru_cache(None)
def _seg_sum_fn(T, ACC, D):
    RPT = ACC // NS
    ZR = min(RPT, 160)
    NCOPY = RPT // ZR

    def body(tab_h, gidx_h, sidx_h, out_h, gix, six, rows, zbuf, acc, sem):
        cid = lax.axis_index("c")
        sid = lax.axis_index("s")
        wid = sid * NC + cid
        pltpu.sync_copy(gidx_h.at[wid], gix)
        pltpu.sync_copy(sidx_h.at[wid], six)
        _zero_fill(zbuf, ZR, D)
        r0 = sid * RPT
        for k in range(NCOPY):
            pltpu.sync_copy(zbuf, acc.at[pl.ds(r0 + k * ZR, ZR)])
        plsc.subcore_barrier()

        def step(j, _):
            pltpu.async_copy(tab_h.at[gix.at[j]], rows, sem).wait()
            pltpu.sync_copy(rows, acc.at[six.at[j]], add=True)
            return 0

        lax.fori_loop(0, NCHUNK, step, 0)
        plsc.subcore_barrier()
        pltpu.sync_copy(acc.at[pl.ds(r0, RPT)], out_h.at[cid, pl.ds(r0, RPT)])

    return pl.kernel(
        body,
        out_type=jax.ShapeDtypeStruct((NC, ACC, D), jnp.float32),
        mesh=_mesh(),
        compiler_params=pltpu.CompilerParams(use_tc_tiling_on_sc=False, needs_layout_passes=False),
        scratch_types=[
            pltpu.VMEM((NCHUNK, CHUNK), jnp.int32),
            pltpu.VMEM((NCHUNK, CHUNK), jnp.int32),
            pltpu.VMEM((CHUNK, D), jnp.float32),
            pltpu.VMEM((ZR, D), jnp.float32),
            pltpu.VMEM_SHARED((ACC, D), jnp.float32),
            pltpu.SemaphoreType.DMA,
        ],
    )


@functools.lru_cache(None)
def _edge_dot_fn(D):
    def body(ta_h, tb_h, ga_h, gb_h, out_h, gia, gib, ra, rb, sv, sema, semb):
        cid = lax.axis_index("c")
        sid = lax.axis_index("s")
        wid = sid * NC + cid
        pltpu.sync_copy(ga_h.at[wid], gia)
        pltpu.sync_copy(gb_h.at[wid], gib)

        lanes = lax.iota(jnp.int32, 16)

        def chunk(j, _):
            da = pltpu.async_copy(ta_h.at[gia.at[j]], ra, sema)
            db = pltpu.async_copy(tb_h.at[gib.at[j]], rb, semb)
            da.wait()
            db.wait()

            def group(g, _):
                evec = g * 16 + lanes
                acc = jnp.zeros((16,), jnp.float32)
                for c in range(D):
                    col = jnp.full((16,), c, jnp.int32)
                    acc = acc + plsc.load_gather(ra, [evec, col]) * plsc.load_gather(rb, [evec, col])
                sv[pl.ds(j * CHUNK + g * 16, 16)] = acc
                return 0

            lax.fori_loop(0, CHUNK // 16, group, 0)
            return 0

        lax.fori_loop(0, NCHUNK, chunk, 0)
        pltpu.sync_copy(sv, out_h.at[wid])

    return pl.kernel(
        body,
        out_type=jax.ShapeDtypeStruct((NW, PAD_EPW), jnp.float32),
        mesh=_mesh(),
        compiler_params=pltpu.CompilerParams(use_tc_tiling_on_sc=False, needs_layout_passes=False),
        scratch_types=[
            pltpu.VMEM((NCHUNK, CHUNK), jnp.int32),
            pltpu.VMEM((NCHUNK, CHUNK), jnp.int32),
            pltpu.VMEM((CHUNK, D), jnp.float32),
            pltpu.VMEM((CHUNK, D), jnp.float32),
            pltpu.VMEM((PAD_EPW,), jnp.float32),
            pltpu.SemaphoreType.DMA,
            pltpu.SemaphoreType.DMA,
        ],
    )


@functools.lru_cache(None)
def _wscatter_fn(T, ACC, D):
    RPT = ACC // NS
    ZR = min(RPT, 160)
    NCOPY = RPT // ZR

    def body(tab_h, w_h, gidx_h, sidx_h, num_h, den_h,
             gix, six, wv, rows, zbuf, zvec, acc, dac, sem):
        cid = lax.axis_index("c")
        sid = lax.axis_index("s")
        wid = sid * NC + cid
        pltpu.sync_copy(gidx_h.at[wid], gix)
        pltpu.sync_copy(sidx_h.at[wid], six)
        pltpu.sync_copy(w_h.at[wid], wv)
        _zero_fill(zbuf, ZR, D)
        _zero_fill_1d(zvec, RPT)
        r0 = sid * RPT
        for k in range(NCOPY):
            pltpu.sync_copy(zbuf, acc.at[pl.ds(r0 + k * ZR, ZR)])
        pltpu.sync_copy(zvec, dac.at[pl.ds(r0, RPT)])
        plsc.subcore_barrier()

        def chunk(j, _):
            pltpu.async_copy(tab_h.at[gix.at[j]], rows, sem).wait()

            def edge(e, _):
                widx = jnp.full((16,), j * CHUNK + e, jnp.int32)
                wb = plsc.load_gather(wv, [widx])
                for c in range(D // 16):
                    rows[e, pl.ds(c * 16, 16)] = rows[e, pl.ds(c * 16, 16)] * wb
                return 0

            lax.fori_loop(0, CHUNK, edge, 0)
            pltpu.sync_copy(rows, acc.at[six.at[j]], add=True)
            pltpu.sync_copy(wv.at[pl.ds(j * CHUNK, CHUNK)], dac.at[six.at[j]], add=True)
            return 0

        lax.fori_loop(0, NCHUNK, chunk, 0)
        plsc.subcore_barrier()
        pltpu.sync_copy(acc.at[pl.ds(r0, RPT)], num_h.at[cid, pl.ds(r0, RPT)])
        pltpu.sync_copy(dac.at[pl.ds(r0, RPT)], den_h.at[cid, pl.ds(r0, RPT)])

    return pl.kernel(
        body,
        out_type=(
            jax.ShapeDtypeStruct((NC, ACC, D), jnp.float32),
            jax.ShapeDtypeStruct((NC, ACC), jnp.float32),
        ),
        mesh=_mesh(),
        compiler_params=pltpu.CompilerParams(use_tc_tiling_on_sc=False, needs_layout_passes=False),
        scratch_types=[
            pltpu.VMEM((NCHUNK, CHUNK), jnp.int32),
            pltpu.VMEM((NCHUNK, CHUNK), jnp.int32),
            pltpu.VMEM((PAD_EPW,), jnp.float32),
            pltpu.VMEM((CHUNK, D), jnp.float32),
            pltpu.VMEM((ZR, D), jnp.float32),
            pltpu.VMEM((RPT,), jnp.float32),
            pltpu.VMEM_SHARED((ACC, D), jnp.float32),
            pltpu.VMEM_SHARED((ACC,), jnp.float32),
            pltpu.SemaphoreType.DMA,
        ],
    )


@functools.lru_cache(None)
def _final_edge_fn(D):
    def body(ta_h, tb_h, w2_h, ga_h, gb_h, out_h, gia, gib, w2v, ra, rb, sv, sema, semb):
        cid = lax.axis_index("c")
        sid = lax.axis_index("s")
        wid = sid * NC + cid
        pltpu.sync_copy(ga_h.at[wid], gia)
        pltpu.sync_copy(gb_h.at[wid], gib)
        pltpu.sync_copy(w2_h, w2v)
        lanes = lax.iota(jnp.int32, 16)

        def chunk(j, _):
            da = pltpu.async_copy(ta_h.at[gia.at[j]], ra, sema)
            db = pltpu.async_copy(tb_h.at[gib.at[j]], rb, semb)
            da.wait()
            db.wait()

            def group(g, _):
                evec = g * 16 + lanes
                acc = jnp.zeros((16,), jnp.float32)
                for c in range(D):
                    col = jnp.full((16,), c, jnp.int32)
                    hcol = jnp.maximum(
                        plsc.load_gather(ra, [evec, col]) + plsc.load_gather(rb, [evec, col]),
                        0.0,
                    )
                    acc = acc + hcol * w2v[c, :]
                sv[pl.ds(j * CHUNK + g * 16, 16)] = acc
                return 0

            lax.fori_loop(0, CHUNK // 16, group, 0)
            return 0

        lax.fori_loop(0, NCHUNK, chunk, 0)
        pltpu.sync_copy(sv, out_h.at[wid])

    return pl.kernel(
        body,
        out_type=jax.ShapeDtypeStruct((NW, PAD_EPW), jnp.float32),
        mesh=_mesh(),
        compiler_params=pltpu.CompilerParams(use_tc_tiling_on_sc=False, needs_layout_passes=False),
        scratch_types=[
            pltpu.VMEM((NCHUNK, CHUNK), jnp.int32),
            pltpu.VMEM((NCHUNK, CHUNK), jnp.int32),
            pltpu.VMEM((D, 16), jnp.float32),
            pltpu.VMEM((CHUNK, D), jnp.float32),
            pltpu.VMEM((CHUNK, D), jnp.float32),
            pltpu.VMEM((PAD_EPW,), jnp.float32),
            pltpu.SemaphoreType.DMA,
            pltpu.SemaphoreType.DMA,
        ],
    )


def _dot(a, b):
    return jnp.dot(a, b, preferred_element_type=jnp.float32)


def _tc_pre(xd_ref, xp_ref, ed_ref, wnd_ref, wnp_ref, wed_ref, h_ref, hp_ref, ew_ref):
    h_ref[...] = _dot(xd_ref[...], wnd_ref[...])
    hp_ref[...] = _dot(xp_ref[...], wnp_ref[...])
    ew_ref[...] = _dot(ed_ref[...], wed_ref[...])


def _tc_rcnt(ci1_ref, ci0_ref, cp1_ref, cp0_ref, ri1_ref, ri0_ref, rp1_ref, rp0_ref):
    ri1_ref[...] = 1.0 / jnp.maximum(ci1_ref[0, :M_D, 0:1] + ci1_ref[1, :M_D, 0:1], 1.0)
    ri0_ref[...] = 1.0 / jnp.maximum(ci0_ref[0, :N_D, 0:1] + ci0_ref[1, :N_D, 0:1], 1.0)
    rp1_ref[...] = 1.0 / jnp.maximum(cp1_ref[0, :M_P, 0:1] + cp1_ref[1, :M_P, 0:1], 1.0)
    rp0_ref[...] = 1.0 / jnp.maximum(cp0_ref[0, :N_P, 0:1] + cp0_ref[1, :N_P, 0:1], 1.0)


def _tc_mid(ae_ref, ri1_ref, ew_ref, ap_ref, rp1_ref, eh_ref, pe_ref):
    ae = ae_ref[0, :M_D, :] + ae_ref[1, :M_D, :]
    eh_ref[...] = jnp.maximum(ae * ri1_ref[...] + ew_ref[...], 0.0)
    ap = ap_ref[0, :M_P, :] + ap_ref[1, :M_P, :]
    pe_ref[...] = jnp.maximum(ap * rp1_ref[...], 0.0)


def _tc_x(s_ref, r_ref, base_ref, x_ref):
    x_ref[...] = jnp.maximum(
        (s_ref[0, :N_D, :] + s_ref[1, :N_D, :]) * r_ref[...] + base_ref[...], 0.0)


def _tc_qkv3(x_ref, wq_ref, wk_ref, wv_ref, q_ref, k_ref, v_ref):
    x = x_ref[...]
    q_ref[...] = _dot(x, wq_ref[...]) * SCALE
    k_ref[...] = _dot(x, wk_ref[...])
    v_ref[...] = _dot(x, wv_ref[...])


def _tc_exp(sdp_ref, spd_ref, edp_ref, epd_ref):
    m1 = jnp.max(sdp_ref[...])
    edp_ref[...] = jnp.exp(sdp_ref[...] - m1)
    m2 = jnp.max(spd_ref[...])
    epd_ref[...] = jnp.exp(spd_ref[...] - m2)


def _tc_post1(x_ref, n_ref, d_ref, wb_ref, ox_ref):
    agg = (n_ref[0, :N_D, :] + n_ref[1, :N_D, :]) / (
        d_ref[0, :N_D, :] + d_ref[1, :N_D, :] + 1e-9)
    ox_ref[...] = _dot(x_ref[...] + agg, wb_ref[...])


def _tc_ab(xd_ref, xp_ref, w1_ref, b1_ref, a_ref, b_ref):
    a_ref[...] = _dot(xd_ref[...], w1_ref[:DIN, :]) + b1_ref[...]
    b_ref[...] = _dot(xp_ref[...], w1_ref[DIN:, :])


def _sds(shape):
    return jax.ShapeDtypeStruct(shape, jnp.float32)


def _prep_idx(idx, pad):
    x = idx.reshape(NW, EPW)
    x = jnp.pad(x, ((0, 0), (0, PAD_EPW - EPW)), constant_values=pad)
    return x.reshape(NW, NCHUNK, CHUNK)


def kernel(x_drug, edge_drug, inc_drug, x_prot, prot_inc, dp_edge_idx,
           Wn_d, We_d, Wn_p, Wbd, Wbp, Wq_d, Wk_p, Wv_p, Wq_p, Wk_d, Wv_d,
           W1, b1, W2):
    g_i0 = _prep_idx(inc_drug[0], 0)
    s_i1 = _prep_idx(inc_drug[1], M_D)
    g_i1 = _prep_idx(inc_drug[1], 0)
    s_i0 = _prep_idx(inc_drug[0], N_D)
    g_p0 = _prep_idx(prot_inc[0], 0)
    s_p1 = _prep_idx(prot_inc[1], M_P)
    g_p1 = _prep_idx(prot_inc[1], 0)
    s_p0 = _prep_idx(prot_inc[0], N_P)
    g_src = _prep_idx(dp_edge_idx[0], 0)
    s_src = _prep_idx(dp_edge_idx[0], N_D)
    g_dst = _prep_idx(dp_edge_idx[1], 0)
    s_dst = _prep_idx(dp_edge_idx[1], N_P)

    seg_nd_md = _seg_sum_fn(N_D, ACC_M, HID)
    seg_md_nd = _seg_sum_fn(M_D, ACC_N, HID)
    seg_cnt_m = _seg_sum_fn(16, ACC_M, HID)
    seg_cnt_n = _seg_sum_fn(16, ACC_N, HID)
    edot = _edge_dot_fn(HID)
    wsc = _wscatter_fn(N_D, ACC_N, HID)
    fedge = _final_edge_fn(HID)

    ones_tab = jnp.ones((16, HID), jnp.float32)
    zero_idx = jnp.zeros((NW, NCHUNK, CHUNK), jnp.int32)
    c_i1 = seg_cnt_m(ones_tab, zero_idx, s_i1)
    c_i0 = seg_cnt_n(ones_tab, zero_idx, s_i0)
    c_p1 = seg_cnt_m(ones_tab, zero_idx, s_p1)
    c_p0 = seg_cnt_n(ones_tab, zero_idx, s_p0)
    ri1, ri0, rp1, rp0 = pl.pallas_call(
        _tc_rcnt,
        out_shape=(_sds((M_D, 1)), _sds((N_D, 1)), _sds((M_P, 1)), _sds((N_P, 1))),
    )(c_i1, c_i0, c_p1, c_p0)

    for i in range(ROUNDS):
        h, hp, ew = pl.pallas_call(
            _tc_pre,
            out_shape=(_sds((N_D, HID)), _sds((N_P, HID)), _sds((M_D, HID))),
        )(x_drug, x_prot, edge_drug, Wn_d[i], Wn_p[i], We_d[i])

        aggE = seg_nd_md(h, g_i0, s_i1)
        aggP = seg_nd_md(hp, g_p0, s_p1)
        e_h, pe = pl.pallas_call(
            _tc_mid,
            out_shape=(_sds((M_D, HID)), _sds((M_P, HID))),
        )(aggE, ri1, ew, aggP, rp1)

        sumD = seg_md_nd(e_h, g_i1, s_i0)
        sumP = seg_md_nd(pe, g_p1, s_p0)
        xd = pl.pallas_call(_tc_x, out_shape=_sds((N_D, HID)))(sumD, ri0, h)
        xp = pl.pallas_call(_tc_x, out_shape=_sds((N_P, HID)))(sumP, rp0, hp)
        qd, kd, vd = pl.pallas_call(
            _tc_qkv3, out_shape=tuple(_sds((N_D, HID)) for _ in range(3))
        )(xd, Wq_d, Wk_d, Wv_d)
        qp, kp, vp = pl.pallas_call(
            _tc_qkv3, out_shape=tuple(_sds((N_P, HID)) for _ in range(3))
        )(xp, Wq_p, Wk_p, Wv_p)

        s_dp = edot(qd, kp, g_src, g_dst)
        s_pd = edot(qp, kd, g_dst, g_src)
        ex_dp, ex_pd = pl.pallas_call(
            _tc_exp,
            out_shape=(_sds((NW, PAD_EPW)), _sds((NW, PAD_EPW))),
        )(s_dp, s_pd)

        num_d, den_d = wsc(vp, ex_dp, g_dst, s_src)
        num_p, den_p = wsc(vd, ex_pd, g_src, s_dst)
        x_drug = pl.pallas_call(_tc_post1, out_shape=_sds((N_D, DIN)))(
            xd, num_d, den_d.reshape(NC, ACC_N, 1), Wbd[i])
        x_prot = pl.pallas_call(_tc_post1, out_shape=_sds((N_P, DIN)))(
            xp, num_p, den_p.reshape(NC, ACC_N, 1), Wbp[i])

    A, B = pl.pallas_call(
        _tc_ab,
        out_shape=(_sds((N_D, HID)), _sds((N_P, HID))),
    )(x_drug, x_prot, W1, b1.reshape(1, HID))

    w2b = jnp.broadcast_to(W2.reshape(HID, 1), (HID, 16))
    lg = fedge(A, B, w2b, g_src, g_dst)
    return lg[:, :EPW].reshape(E)


# speedup vs baseline: 5.3703x; 3.4789x over previous
import functools

import jax
import jax.numpy as jnp
from jax import lax
from jax.experimental import pallas as pl
from jax.experimental.pallas import tpu as pltpu
from jax.experimental.pallas import tpu_sc as plsc

N_D = 10000
M_D = 2500
N_P = 10000
M_P = 2500
E = 320000
DIN = 128
HID = 64
ROUNDS = 3
SCALE = 1.0 / (HID ** 0.5)

NC = 2
NS = 16
NW = NC * NS
CHUNK = 128
EPW = E // NW
NCHUNK = (EPW + CHUNK - 1) // CHUNK
PAD_EPW = NCHUNK * CHUNK


def _acc_rows(n):
    return ((n + 1 + 255) // 256) * 256


ACC_N = _acc_rows(N_D)
ACC_M = _acc_rows(M_D)


def _mesh():
    return plsc.VectorSubcoreMesh(
        core_axis_name="c", subcore_axis_name="s", num_cores=NC, num_subcores=NS
    )


def _zero_fill(zbuf, nrow, ncol):
    zf = jnp.zeros((16,), jnp.float32)
    cpr = ncol // 16

    def zb(i, _):
        r = i // cpr
        c = (i % cpr) * 16
        zbuf[r, pl.ds(c, 16)] = zf
        return 0

    lax.fori_loop(0, nrow * cpr, zb, 0)


def _zero_fill_1d(zvec, n):
    zf = jnp.zeros((16,), jnp.float32)

    def zb(i, _):
        zvec[pl.ds(i * 16, 16)] = zf
        return 0

    lax.fori_loop(0, n // 16, zb, 0)


@functools.lru_cache(None)
def _seg_sum_fn(T, ACC, D):
    RPT = ACC // NS
    ZR = min(RPT, 160)
    NCOPY = RPT // ZR

    def body(tab_h, gidx_h, sidx_h, out_h, gix, six, rows, zbuf, acc, sem):
        cid = lax.axis_index("c")
        sid = lax.axis_index("s")
        wid = sid * NC + cid
        pltpu.sync_copy(gidx_h.at[wid], gix)
        pltpu.sync_copy(sidx_h.at[wid], six)
        _zero_fill(zbuf, ZR, D)
        r0 = sid * RPT
        for k in range(NCOPY):
            pltpu.sync_copy(zbuf, acc.at[pl.ds(r0 + k * ZR, ZR)])
        plsc.subcore_barrier()

        def step(j, _):
            pltpu.async_copy(tab_h.at[gix.at[j]], rows, sem).wait()
            pltpu.sync_copy(rows, acc.at[six.at[j]], add=True)
            return 0

        lax.fori_loop(0, NCHUNK, step, 0)
        plsc.subcore_barrier()
        pltpu.sync_copy(acc.at[pl.ds(r0, RPT)], out_h.at[cid, pl.ds(r0, RPT)])

    return pl.kernel(
        body,
        out_type=jax.ShapeDtypeStruct((NC, ACC, D), jnp.float32),
        mesh=_mesh(),
        compiler_params=pltpu.CompilerParams(use_tc_tiling_on_sc=False, needs_layout_passes=False),
        scratch_types=[
            pltpu.VMEM((NCHUNK, CHUNK), jnp.int32),
            pltpu.VMEM((NCHUNK, CHUNK), jnp.int32),
            pltpu.VMEM((CHUNK, D), jnp.float32),
            pltpu.VMEM((ZR, D), jnp.float32),
            pltpu.VMEM_SHARED((ACC, D), jnp.float32),
            pltpu.SemaphoreType.DMA,
        ],
    )


@functools.lru_cache(None)
def _count_fn(ACC):
    RPT = ACC // NS
    CD = 16

    def body(sidx_h, out_h, six, ones_v, zbuf, acc):
        cid = lax.axis_index("c")
        sid = lax.axis_index("s")
        wid = sid * NC + cid
        pltpu.sync_copy(sidx_h.at[wid], six)
        one = jnp.ones((16,), jnp.float32)

        def ob(i, _):
            ones_v[i, pl.ds(0, 16)] = one
            return 0

        lax.fori_loop(0, CHUNK, ob, 0)
        _zero_fill(zbuf, RPT, CD)
        r0 = sid * RPT
        pltpu.sync_copy(zbuf, acc.at[pl.ds(r0, RPT)])
        plsc.subcore_barrier()

        def step(j, _):
            pltpu.sync_copy(ones_v, acc.at[six.at[j]], add=True)
            return 0

        lax.fori_loop(0, NCHUNK, step, 0)
        plsc.subcore_barrier()
        pltpu.sync_copy(acc.at[pl.ds(r0, RPT)], out_h.at[cid, pl.ds(r0, RPT)])

    return pl.kernel(
        body,
        out_type=jax.ShapeDtypeStruct((NC, ACC, CD), jnp.float32),
        mesh=_mesh(),
        compiler_params=pltpu.CompilerParams(use_tc_tiling_on_sc=False, needs_layout_passes=False),
        scratch_types=[
            pltpu.VMEM((NCHUNK, CHUNK), jnp.int32),
            pltpu.VMEM((CHUNK, CD), jnp.float32),
            pltpu.VMEM((RPT, CD), jnp.float32),
            pltpu.VMEM_SHARED((ACC, CD), jnp.float32),
        ],
    )


@functools.lru_cache(None)
def _edge_dot_fn(D):
    def body(ta_h, tb_h, ga_h, gb_h, out_h, gia, gib, ra, rb, sv, sema, semb):
        cid = lax.axis_index("c")
        sid = lax.axis_index("s")
        wid = sid * NC + cid
        pltpu.sync_copy(ga_h.at[wid], gia)
        pltpu.sync_copy(gb_h.at[wid], gib)

        lanes = lax.iota(jnp.int32, 16)

        def chunk(j, _):
            da = pltpu.async_copy(ta_h.at[gia.at[j]], ra, sema)
            db = pltpu.async_copy(tb_h.at[gib.at[j]], rb, semb)
            da.wait()
            db.wait()

            def group(g, _):
                evec = g * 16 + lanes
                acc = jnp.zeros((16,), jnp.float32)
                for c in range(D):
                    col = jnp.full((16,), c, jnp.int32)
                    acc = acc + plsc.load_gather(ra, [evec, col]) * plsc.load_gather(rb, [evec, col])
                sv[pl.ds(j * CHUNK + g * 16, 16)] = acc
                return 0

            lax.fori_loop(0, CHUNK // 16, group, 0)
            return 0

        lax.fori_loop(0, NCHUNK, chunk, 0)
        pltpu.sync_copy(sv, out_h.at[wid])

    return pl.kernel(
        body,
        out_type=jax.ShapeDtypeStruct((NW, PAD_EPW), jnp.float32),
        mesh=_mesh(),
        compiler_params=pltpu.CompilerParams(use_tc_tiling_on_sc=False, needs_layout_passes=False),
        scratch_types=[
            pltpu.VMEM((NCHUNK, CHUNK), jnp.int32),
            pltpu.VMEM((NCHUNK, CHUNK), jnp.int32),
            pltpu.VMEM((CHUNK, D), jnp.float32),
            pltpu.VMEM((CHUNK, D), jnp.float32),
            pltpu.VMEM((PAD_EPW,), jnp.float32),
            pltpu.SemaphoreType.DMA,
            pltpu.SemaphoreType.DMA,
        ],
    )


@functools.lru_cache(None)
def _wscatter_fn(T, ACC, D):
    RPT = ACC // NS
    ZR = min(RPT, 160)
    NCOPY = RPT // ZR

    def body(tab_h, w_h, gidx_h, sidx_h, num_h, den_h,
             gix, six, wv, rows, zbuf, zvec, acc, dac, sem):
        cid = lax.axis_index("c")
        sid = lax.axis_index("s")
        wid = sid * NC + cid
        pltpu.sync_copy(gidx_h.at[wid], gix)
        pltpu.sync_copy(sidx_h.at[wid], six)
        pltpu.sync_copy(w_h.at[wid], wv)
        _zero_fill(zbuf, ZR, D)
        _zero_fill_1d(zvec, RPT)
        r0 = sid * RPT
        for k in range(NCOPY):
            pltpu.sync_copy(zbuf, acc.at[pl.ds(r0 + k * ZR, ZR)])
        pltpu.sync_copy(zvec, dac.at[pl.ds(r0, RPT)])
        plsc.subcore_barrier()

        def chunk(j, _):
            pltpu.async_copy(tab_h.at[gix.at[j]], rows, sem).wait()

            def edge(e, _):
                widx = jnp.full((16,), j * CHUNK + e, jnp.int32)
                wb = plsc.load_gather(wv, [widx])
                for c in range(D // 16):
                    rows[e, pl.ds(c * 16, 16)] = rows[e, pl.ds(c * 16, 16)] * wb
                return 0

            lax.fori_loop(0, CHUNK, edge, 0)
            pltpu.sync_copy(rows, acc.at[six.at[j]], add=True)
            pltpu.sync_copy(wv.at[pl.ds(j * CHUNK, CHUNK)], dac.at[six.at[j]], add=True)
            return 0

        lax.fori_loop(0, NCHUNK, chunk, 0)
        plsc.subcore_barrier()
        pltpu.sync_copy(acc.at[pl.ds(r0, RPT)], num_h.at[cid, pl.ds(r0, RPT)])
        pltpu.sync_copy(dac.at[pl.ds(r0, RPT)], den_h.at[cid, pl.ds(r0, RPT)])

    return pl.kernel(
        body,
        out_type=(
            jax.ShapeDtypeStruct((NC, ACC, D), jnp.float32),
            jax.ShapeDtypeStruct((NC, ACC), jnp.float32),
        ),
        mesh=_mesh(),
        compiler_params=pltpu.CompilerParams(use_tc_tiling_on_sc=False, needs_layout_passes=False),
        scratch_types=[
            pltpu.VMEM((NCHUNK, CHUNK), jnp.int32),
            pltpu.VMEM((NCHUNK, CHUNK), jnp.int32),
            pltpu.VMEM((PAD_EPW,), jnp.float32),
            pltpu.VMEM((CHUNK, D), jnp.float32),
            pltpu.VMEM((ZR, D), jnp.float32),
            pltpu.VMEM((RPT,), jnp.float32),
            pltpu.VMEM_SHARED((ACC, D), jnp.float32),
            pltpu.VMEM_SHARED((ACC,), jnp.float32),
            pltpu.SemaphoreType.DMA,
        ],
    )


@functools.lru_cache(None)
def _final_edge_fn(D):
    def body(ta_h, tb_h, w2_h, ga_h, gb_h, out_h, gia, gib, w2v, ra, rb, sv, sema, semb):
        cid = lax.axis_index("c")
        sid = lax.axis_index("s")
        wid = sid * NC + cid
        pltpu.sync_copy(ga_h.at[wid], gia)
        pltpu.sync_copy(gb_h.at[wid], gib)
        pltpu.sync_copy(w2_h, w2v)
        lanes = lax.iota(jnp.int32, 16)

        def chunk(j, _):
            da = pltpu.async_copy(ta_h.at[gia.at[j]], ra, sema)
            db = pltpu.async_copy(tb_h.at[gib.at[j]], rb, semb)
            da.wait()
            db.wait()

            def group(g, _):
                evec = g * 16 + lanes
                acc = jnp.zeros((16,), jnp.float32)
                for c in range(D):
                    col = jnp.full((16,), c, jnp.int32)
                    hcol = jnp.maximum(
                        plsc.load_gather(ra, [evec, col]) + plsc.load_gather(rb, [evec, col]),
                        0.0,
                    )
                    acc = acc + hcol * w2v[c, :]
                sv[pl.ds(j * CHUNK + g * 16, 16)] = acc
                return 0

            lax.fori_loop(0, CHUNK // 16, group, 0)
            return 0

        lax.fori_loop(0, NCHUNK, chunk, 0)
        pltpu.sync_copy(sv, out_h.at[wid])

    return pl.kernel(
        body,
        out_type=jax.ShapeDtypeStruct((NW, PAD_EPW), jnp.float32),
        mesh=_mesh(),
        compiler_params=pltpu.CompilerParams(use_tc_tiling_on_sc=False, needs_layout_passes=False),
        scratch_types=[
            pltpu.VMEM((NCHUNK, CHUNK), jnp.int32),
            pltpu.VMEM((NCHUNK, CHUNK), jnp.int32),
            pltpu.VMEM((D, 16), jnp.float32),
            pltpu.VMEM((CHUNK, D), jnp.float32),
            pltpu.VMEM((CHUNK, D), jnp.float32),
            pltpu.VMEM((PAD_EPW,), jnp.float32),
            pltpu.SemaphoreType.DMA,
            pltpu.SemaphoreType.DMA,
        ],
    )


def _dot(a, b):
    return jnp.dot(a, b, preferred_element_type=jnp.float32)


def _tc_pre(xd_ref, xp_ref, ed_ref, wnd_ref, wnp_ref, wed_ref, h_ref, hp_ref, ew_ref):
    h_ref[...] = _dot(xd_ref[...], wnd_ref[...])
    hp_ref[...] = _dot(xp_ref[...], wnp_ref[...])
    ew_ref[...] = _dot(ed_ref[...], wed_ref[...])


def _tc_rcnt(ci1_ref, ci0_ref, cp1_ref, cp0_ref, ri1_ref, ri0_ref, rp1_ref, rp0_ref):
    ri1_ref[...] = 1.0 / jnp.maximum(ci1_ref[0, :M_D, 0:1] + ci1_ref[1, :M_D, 0:1], 1.0)
    ri0_ref[...] = 1.0 / jnp.maximum(ci0_ref[0, :N_D, 0:1] + ci0_ref[1, :N_D, 0:1], 1.0)
    rp1_ref[...] = 1.0 / jnp.maximum(cp1_ref[0, :M_P, 0:1] + cp1_ref[1, :M_P, 0:1], 1.0)
    rp0_ref[...] = 1.0 / jnp.maximum(cp0_ref[0, :N_P, 0:1] + cp0_ref[1, :N_P, 0:1], 1.0)


def _tc_mid(ae_ref, ri1_ref, ew_ref, ap_ref, rp1_ref, eh_ref, pe_ref):
    ae = ae_ref[0, :M_D, :] + ae_ref[1, :M_D, :]
    eh_ref[...] = jnp.maximum(ae * ri1_ref[...] + ew_ref[...], 0.0)
    ap = ap_ref[0, :M_P, :] + ap_ref[1, :M_P, :]
    pe_ref[...] = jnp.maximum(ap * rp1_ref[...], 0.0)


def _tc_x(s_ref, r_ref, base_ref, x_ref):
    x_ref[...] = jnp.maximum(
        (s_ref[0, :N_D, :] + s_ref[1, :N_D, :]) * r_ref[...] + base_ref[...], 0.0)


def _tc_qkv3(x_ref, wq_ref, wk_ref, wv_ref, q_ref, k_ref, v_ref):
    x = x_ref[...]
    q_ref[...] = _dot(x, wq_ref[...]) * SCALE
    k_ref[...] = _dot(x, wk_ref[...])
    v_ref[...] = _dot(x, wv_ref[...])


def _tc_exp(sdp_ref, spd_ref, edp_ref, epd_ref):
    m1 = jnp.max(sdp_ref[...])
    edp_ref[...] = jnp.exp(sdp_ref[...] - m1)
    m2 = jnp.max(spd_ref[...])
    epd_ref[...] = jnp.exp(spd_ref[...] - m2)


def _tc_post1(x_ref, n_ref, d_ref, wb_ref, ox_ref):
    agg = (n_ref[0, :N_D, :] + n_ref[1, :N_D, :]) / (
        d_ref[0, :N_D, :] + d_ref[1, :N_D, :] + 1e-9)
    ox_ref[...] = _dot(x_ref[...] + agg, wb_ref[...])


def _tc_ab(xd_ref, xp_ref, w1_ref, b1_ref, a_ref, b_ref):
    a_ref[...] = _dot(xd_ref[...], w1_ref[:DIN, :]) + b1_ref[...]
    b_ref[...] = _dot(xp_ref[...], w1_ref[DIN:, :])


def _sds(shape):
    return jax.ShapeDtypeStruct(shape, jnp.float32)


def _prep_idx(idx, pad):
    x = idx.reshape(NW, EPW)
    x = jnp.pad(x, ((0, 0), (0, PAD_EPW - EPW)), constant_values=pad)
    return x.reshape(NW, NCHUNK, CHUNK)


def kernel(x_drug, edge_drug, inc_drug, x_prot, prot_inc, dp_edge_idx,
           Wn_d, We_d, Wn_p, Wbd, Wbp, Wq_d, Wk_p, Wv_p, Wq_p, Wk_d, Wv_d,
           W1, b1, W2):
    g_i0 = _prep_idx(inc_drug[0], 0)
    s_i1 = _prep_idx(inc_drug[1], M_D)
    g_i1 = _prep_idx(inc_drug[1], 0)
    s_i0 = _prep_idx(inc_drug[0], N_D)
    g_p0 = _prep_idx(prot_inc[0], 0)
    s_p1 = _prep_idx(prot_inc[1], M_P)
    g_p1 = _prep_idx(prot_inc[1], 0)
    s_p0 = _prep_idx(prot_inc[0], N_P)
    g_src = _prep_idx(dp_edge_idx[0], 0)
    s_src = _prep_idx(dp_edge_idx[0], N_D)
    g_dst = _prep_idx(dp_edge_idx[1], 0)
    s_dst = _prep_idx(dp_edge_idx[1], N_P)

    seg_nd_md = _seg_sum_fn(N_D, ACC_M, HID)
    seg_md_nd = _seg_sum_fn(M_D, ACC_N, HID)
    cnt_m = _count_fn(ACC_M)
    cnt_n = _count_fn(ACC_N)
    edot = _edge_dot_fn(HID)
    wsc = _wscatter_fn(N_D, ACC_N, HID)
    fedge = _final_edge_fn(HID)

    c_i1 = cnt_m(s_i1)
    c_i0 = cnt_n(s_i0)
    c_p1 = cnt_m(s_p1)
    c_p0 = cnt_n(s_p0)
    ri1, ri0, rp1, rp0 = pl.pallas_call(
        _tc_rcnt,
        out_shape=(_sds((M_D, 1)), _sds((N_D, 1)), _sds((M_P, 1)), _sds((N_P, 1))),
    )(c_i1, c_i0, c_p1, c_p0)

    for i in range(ROUNDS):
        h, hp, ew = pl.pallas_call(
            _tc_pre,
            out_shape=(_sds((N_D, HID)), _sds((N_P, HID)), _sds((M_D, HID))),
        )(x_drug, x_prot, edge_drug, Wn_d[i], Wn_p[i], We_d[i])

        aggE = seg_nd_md(h, g_i0, s_i1)
        aggP = seg_nd_md(hp, g_p0, s_p1)
        e_h, pe = pl.pallas_call(
            _tc_mid,
            out_shape=(_sds((M_D, HID)), _sds((M_P, HID))),
        )(aggE, ri1, ew, aggP, rp1)

        sumD = seg_md_nd(e_h, g_i1, s_i0)
        sumP = seg_md_nd(pe, g_p1, s_p0)
        xd = pl.pallas_call(_tc_x, out_shape=_sds((N_D, HID)))(sumD, ri0, h)
        xp = pl.pallas_call(_tc_x, out_shape=_sds((N_P, HID)))(sumP, rp0, hp)
        qd, kd, vd = pl.pallas_call(
            _tc_qkv3, out_shape=tuple(_sds((N_D, HID)) for _ in range(3))
        )(xd, Wq_d, Wk_d, Wv_d)
        qp, kp, vp = pl.pallas_call(
            _tc_qkv3, out_shape=tuple(_sds((N_P, HID)) for _ in range(3))
        )(xp, Wq_p, Wk_p, Wv_p)

        s_dp = edot(qd, kp, g_src, g_dst)
        s_pd = edot(qp, kd, g_dst, g_src)
        ex_dp, ex_pd = pl.pallas_call(
            _tc_exp,
            out_shape=(_sds((NW, PAD_EPW)), _sds((NW, PAD_EPW))),
        )(s_dp, s_pd)

        num_d, den_d = wsc(vp, ex_dp, g_dst, s_src)
        num_p, den_p = wsc(vd, ex_pd, g_src, s_dst)
        x_drug = pl.pallas_call(_tc_post1, out_shape=_sds((N_D, DIN)))(
            xd, num_d, den_d.reshape(NC, ACC_N, 1), Wbd[i])
        x_prot = pl.pallas_call(_tc_post1, out_shape=_sds((N_P, DIN)))(
            xp, num_p, den_p.reshape(NC, ACC_N, 1), Wbp[i])

    A, B = pl.pallas_call(
        _tc_ab,
        out_shape=(_sds((N_D, HID)), _sds((N_P, HID))),
    )(x_drug, x_prot, W1, b1.reshape(1, HID))

    w2b = jnp.broadcast_to(W2.reshape(HID, 1), (HID, 16))
    lg = fedge(A, B, w2b, g_src, g_dst)
    return lg[:, :EPW].reshape(E)
